# baseline (speedup 1.0000x reference)
import functools
import jax
import jax.numpy as jnp
from jax import lax
from jax.experimental import pallas as pl
from jax.experimental.pallas import tpu as pltpu
from jax.experimental.pallas import tpu_sc as plsc

NC, NS, NW, LANES = 2, 16, 32, 16


def kernel(tokens, embedding):
    B, L = tokens.shape
    V, D = embedding.shape
    BPW = B // NW
    RND = 128
    SUP = 512
    NSUB = SUP // RND
    ROWS_PER_RND = RND // L
    NROUNDS = (BPW * L) // RND
    NVREG = D // LANES
    VMAIN = (V // SUP) * SUP
    NSUP = VMAIN // SUP
    BASE_CNT = NSUP // NW
    EXTRA = NSUP - BASE_CNT * NW
    SUB_W = RND * D

    tok = tokens.astype(jnp.int32).reshape(NW, NROUNDS, RND)
    table_t = embedding.T
    tail = embedding[VMAIN:, :].reshape(-1)

    mesh = plsc.VectorSubcoreMesh(core_axis_name="c", subcore_axis_name="s")
    params_tc = pltpu.CompilerParams(
        use_tc_tiling_on_sc=True, needs_layout_passes=False)
    params_sc = pltpu.CompilerParams(use_tc_tiling_on_sc=False)

    @functools.partial(
        pl.kernel,
        out_type=jax.ShapeDtypeStruct((V * D,), jnp.float32),
        mesh=mesh,
        compiler_params=params_tc,
        scratch_types=[
            pltpu.VMEM((D, SUP + 1), jnp.float32),
            pltpu.VMEM((D, SUP + 1), jnp.float32),
            pltpu.VMEM((SUB_W,), jnp.float32),
            pltpu.VMEM((SUB_W,), jnp.float32),
            pltpu.SemaphoreType.DMA,
            pltpu.SemaphoreType.DMA,
            pltpu.SemaphoreType.DMA,
            pltpu.SemaphoreType.DMA,
        ],
    )
    def fmt_kernel(tab_hbm, tail_hbm, fmt_hbm, blk0, blk1, buf0, buf1,
                   si0, si1, so0, so1):
        wid = lax.axis_index("s") * NC + lax.axis_index("c")
        nsup = BASE_CNT + (wid < EXTRA).astype(jnp.int32)
        iot = lax.iota(jnp.int32, LANES)
        idxc = [c * LANES + iot for c in range(NVREG)]
        blks = (blk0, blk1)
        bufs = (buf0, buf1)
        sis = (si0, si1)
        sos = (so0, so1)

        @pl.when(wid == NW - 1)
        def _():
            pltpu.sync_copy(tail_hbm, fmt_hbm.at[pl.ds(VMAIN * D, (V - VMAIN) * D)])

        def sbase_of(k):
            return pl.multiple_of((wid + NW * k) * SUP, SUP)

        def issue_in(k, b):
            pltpu.async_copy(tab_hbm.at[:, pl.ds(sbase_of(k), SUP)],
                             blks[b].at[:, pl.ds(0, SUP)], sis[b])

        issue_in(0, 0)

        def pair_body(h, carry):
            for b in range(2):
                k = 2 * h + b

                @pl.when(k < nsup)
                def _():
                    nxt = k + 1

                    @pl.when(nxt < nsup)
                    def _():
                        issue_in(nxt, 1 - b)

                    pltpu.make_async_copy(
                        tab_hbm.at[:, pl.ds(sbase_of(k), SUP)],
                        blks[b].at[:, pl.ds(0, SUP)], sis[b]).wait()

                    blk = blks[b]
                    for sub in range(NSUB):
                        p = sub % 2
                        buf = bufs[p]
                        if sub < 2:
                            @pl.when(k > 0)
                            def _():
                                pltpu.make_async_copy(
                                    buf, fmt_hbm.at[pl.ds(0, SUB_W)],
                                    sos[p]).wait()
                        else:
                            pltpu.make_async_copy(
                                buf, fmt_hbm.at[pl.ds(0, SUB_W)], sos[p]).wait()

                        voff = sub * RND

                        @plsc.parallel_loop(0, RND, unroll=4)
                        def v_body(v):
                            col = jnp.full((LANES,), v, jnp.int32) + voff
                            for c in range(NVREG):
                                vals = plsc.load_gather(blk, [idxc[c], col])
                                buf[pl.ds(v * D + c * LANES, LANES)] = vals

                        pltpu.async_copy(
                            buf,
                            fmt_hbm.at[pl.ds((sbase_of(k) + voff) * D, SUB_W)],
                            sos[p])
            return carry

        lax.fori_loop(0, (BASE_CNT + EXTRA + 1) // 2, pair_body, 0)

        for b in range(2):
            pltpu.make_async_copy(
                bufs[b], fmt_hbm.at[pl.ds(0, SUB_W)], sos[b]).wait()

    NRB = 4

    @functools.partial(
        pl.kernel,
        out_type=jax.ShapeDtypeStruct((B * D,), jnp.float32),
        mesh=mesh,
        compiler_params=params_sc,
        scratch_types=[
            pltpu.VMEM((NROUNDS, RND), jnp.int32),
            pltpu.VMEM((RND, D), jnp.float32),
            pltpu.VMEM((RND, D), jnp.float32),
            pltpu.VMEM((RND, D), jnp.float32),
            pltpu.VMEM((RND, D), jnp.float32),
            pltpu.VMEM((BPW * D,), jnp.float32),
            pltpu.SemaphoreType.DMA,
            pltpu.SemaphoreType.DMA,
            pltpu.SemaphoreType.DMA,
            pltpu.SemaphoreType.DMA,
        ],
    )
    def pool_kernel(tok_hbm, fmt_hbm, out_hbm, idx_v, rb0, rb1, rb2, rb3,
                    out_v, s0, s1, s2, s3):
        wid = lax.axis_index("s") * NC + lax.axis_index("c")
        pltpu.sync_copy(tok_hbm.at[wid], idx_v)
        inv_l = jnp.float32(1.0 / L)
        rbs = (rb0, rb1, rb2, rb3)
        sems = (s0, s1, s2, s3)

        for p in range(NRB - 1):
            pltpu.async_copy(fmt_hbm.at[idx_v.at[p]], rbs[p], sems[p])

        def quad_body(h, carry):
            for b in range(NRB):
                j = NRB * h + b
                nxt = j + NRB - 1
                nb = (b + NRB - 1) % NRB

                @pl.when(nxt < NROUNDS)
                def _():
                    pltpu.async_copy(
                        fmt_hbm.at[idx_v.at[nxt]], rbs[nb], sems[nb])

                pltpu.make_async_copy(
                    fmt_hbm.at[idx_v.at[j]], rbs[b], sems[b]).wait()
                rb = rbs[b]
                for r in range(ROWS_PER_RND):
                    for c in range(NVREG):
                        acc = rb[r * L, pl.ds(c * LANES, LANES)]
                        for k in range(1, L):
                            acc = acc + rb[r * L + k, pl.ds(c * LANES, LANES)]
                        out_v[pl.ds((j * ROWS_PER_RND + r) * D + c * LANES,
                                    LANES)] = acc * inv_l
            return carry

        lax.fori_loop(0, NROUNDS // NRB, quad_body, 0)
        pltpu.sync_copy(out_v, out_hbm.at[pl.ds(wid * BPW * D, BPW * D)])

    fmt = fmt_kernel(table_t, tail)
    return pool_kernel(tok, fmt.reshape(V, D)).reshape(B, D)
